# baseline (speedup 1.0000x reference)
import jax
import jax.numpy as jnp
from jax.experimental import pallas as pl
from jax.experimental.pallas import tpu as pltpu

N = 4096
NFEAT = 128
NHID = 128
NCLASS = 40
BR = 256
NB = N // BR


def _gcn_kernel(x_ref, adj_ref, W1_ref, b1_ref, W2_ref, b2_ref,
                Wout_ref, bout_ref, out_ref, proj_scr, hw_scr):
    l = pl.program_id(0)
    i = pl.program_id(1)

    @pl.when(jnp.logical_and(l == 0, i == 0))
    def _():
        proj_scr[...] = jnp.dot(x_ref[...], W1_ref[...],
                                preferred_element_type=jnp.float32)

    @pl.when(l == 0)
    def _():
        h = jnp.dot(adj_ref[0], proj_scr[...],
                    preferred_element_type=jnp.float32) + b1_ref[...]
        h1 = jnp.maximum(h, 0.0)
        hw_scr[pl.ds(i * BR, BR), :] = jnp.dot(
            h1, W2_ref[...], preferred_element_type=jnp.float32)

    @pl.when(l == 1)
    def _():
        h = jnp.dot(adj_ref[0], hw_scr[...],
                    preferred_element_type=jnp.float32) + b2_ref[...]
        h2 = jnp.maximum(h, 0.0)
        out_ref[...] = jnp.dot(h2, Wout_ref[...],
                               preferred_element_type=jnp.float32) + bout_ref[...]


def kernel(x, adjs, W1, b1, W2, b2, Wout, bout):
    b1r = b1.reshape(1, NHID)
    b2r = b2.reshape(1, NHID)
    boutr = bout.reshape(1, NCLASS)
    return pl.pallas_call(
        _gcn_kernel,
        grid=(2, NB),
        in_specs=[
            pl.BlockSpec((N, NFEAT), lambda l, i: (0, 0)),
            pl.BlockSpec((1, BR, N), lambda l, i: (l, i, 0)),
            pl.BlockSpec((NFEAT, NHID), lambda l, i: (0, 0)),
            pl.BlockSpec((1, NHID), lambda l, i: (0, 0)),
            pl.BlockSpec((NHID, NHID), lambda l, i: (0, 0)),
            pl.BlockSpec((1, NHID), lambda l, i: (0, 0)),
            pl.BlockSpec((NHID, NCLASS), lambda l, i: (0, 0)),
            pl.BlockSpec((1, NCLASS), lambda l, i: (0, 0)),
        ],
        out_specs=pl.BlockSpec((BR, NCLASS), lambda l, i: (i, 0)),
        out_shape=jax.ShapeDtypeStruct((N, NCLASS), jnp.float32),
        scratch_shapes=[
            pltpu.VMEM((N, NHID), jnp.float32),
            pltpu.VMEM((N, NHID), jnp.float32),
        ],
    )(x, adjs, W1, b1r, W2, b2r, Wout, boutr)


# speedup vs baseline: 1.2211x; 1.2211x over previous
import jax
import jax.numpy as jnp
from jax.experimental import pallas as pl
from jax.experimental.pallas import tpu as pltpu

N = 4096
NFEAT = 128
NHID = 128
NCLASS = 40
BR = 512
NB = N // BR


def _gcn_kernel(x_ref, adj_ref, W1_ref, b1_ref, W2_ref, b2_ref,
                Wout_ref, bout_ref, out_ref, proj_scr, hw_scr):
    l = pl.program_id(0)
    i = pl.program_id(1)

    @pl.when(jnp.logical_and(l == 0, i == 0))
    def _():
        proj_scr[...] = jnp.dot(x_ref[...], W1_ref[...],
                                preferred_element_type=jnp.float32)

    @pl.when(l == 0)
    def _():
        a = adj_ref[0]
        h = (jnp.dot(a[:, :2048], proj_scr[:2048, :],
                     preferred_element_type=jnp.float32)
             + jnp.dot(a[:, 2048:], proj_scr[2048:, :],
                       preferred_element_type=jnp.float32)) + b1_ref[...]
        h1 = jnp.maximum(h, 0.0)
        hw_scr[pl.ds(i * BR, BR), :] = jnp.dot(
            h1, W2_ref[...], preferred_element_type=jnp.float32)

    @pl.when(l == 1)
    def _():
        a = adj_ref[0]
        h = (jnp.dot(a[:, :2048], hw_scr[:2048, :],
                     preferred_element_type=jnp.float32)
             + jnp.dot(a[:, 2048:], hw_scr[2048:, :],
                       preferred_element_type=jnp.float32)) + b2_ref[...]
        h2 = jnp.maximum(h, 0.0)
        out_ref[...] = jnp.dot(h2, Wout_ref[...],
                               preferred_element_type=jnp.float32) + bout_ref[...]


def kernel(x, adjs, W1, b1, W2, b2, Wout, bout):
    b1r = b1.reshape(1, NHID)
    b2r = b2.reshape(1, NHID)
    boutr = bout.reshape(1, NCLASS)
    return pl.pallas_call(
        _gcn_kernel,
        grid=(2, NB),
        in_specs=[
            pl.BlockSpec((N, NFEAT), lambda l, i: (0, 0)),
            pl.BlockSpec((1, BR, N), lambda l, i: (l, i, 0)),
            pl.BlockSpec((NFEAT, NHID), lambda l, i: (0, 0)),
            pl.BlockSpec((1, NHID), lambda l, i: (0, 0)),
            pl.BlockSpec((NHID, NHID), lambda l, i: (0, 0)),
            pl.BlockSpec((1, NHID), lambda l, i: (0, 0)),
            pl.BlockSpec((NHID, NCLASS), lambda l, i: (0, 0)),
            pl.BlockSpec((1, NCLASS), lambda l, i: (0, 0)),
        ],
        out_specs=pl.BlockSpec((BR, NCLASS), lambda l, i: (i, 0)),
        out_shape=jax.ShapeDtypeStruct((N, NCLASS), jnp.float32),
        scratch_shapes=[
            pltpu.VMEM((N, NHID), jnp.float32),
            pltpu.VMEM((N, NHID), jnp.float32),
        ],
    )(x, adjs, W1, b1r, W2, b2r, Wout, boutr)
